# baseline (speedup 1.0000x reference)
import functools

import jax
import jax.numpy as jnp
from jax import lax
from jax.experimental import pallas as pl
from jax.experimental.pallas import tpu as pltpu
from jax.experimental.pallas import tpu_sc as plsc

N_NODES = 10000
N_EDGES = 320000
F = 128

NC = 2
NS = 16
NW = NC * NS

CHUNK = 128
NBUF = 2
K0 = 116
K1 = 42
E_PAD = NS * (K0 + K1) * CHUNK
E_SPLIT = NS * K0 * CHUNK
PAD_ROWS = 8

ROWS_MAIN = 624
LAST_START = (NS - 1) * ROWS_MAIN
LAST_ROWS = N_NODES - LAST_START

_sc_mesh = plsc.VectorSubcoreMesh(core_axis_name="c", subcore_axis_name="s")


@functools.partial(
    pl.kernel,
    out_type=jax.ShapeDtypeStruct((NC, N_NODES, F), jnp.float32),
    mesh=_sc_mesh,
    scratch_types=[
        pltpu.VMEM_SHARED((N_NODES + PAD_ROWS, F), jnp.float32),
        [pltpu.VMEM((2, CHUNK), jnp.int32) for _ in range(NBUF)],
        pltpu.VMEM((NBUF, CHUNK, F), jnp.float32),
        [pltpu.SemaphoreType.DMA for _ in range(NBUF)],
        [pltpu.SemaphoreType.DMA for _ in range(NBUF)],
        [pltpu.SemaphoreType.DMA for _ in range(NBUF)],
    ],
)
def _sc_aggregate(x_hbm, idx0_hbm, idx1_hbm, parts_hbm,
                  agg_s, idx_v, rows, sem_i, sem_g, sem_s):
    c = lax.axis_index("c")
    s = lax.axis_index("s")

    start = pl.multiple_of(s * ROWS_MAIN, 8)

    zv = jnp.zeros((16,), jnp.float32)

    def _zrow(r, carry):
        for kk in range(F // 16):
            rows[0, r, pl.ds(kk * 16, 16)] = zv
        return carry

    lax.fori_loop(0, CHUNK, _zrow, 0)

    @pl.when(s < NS - 1)
    def _():
        for i in range(4):
            pltpu.sync_copy(rows.at[0],
                            agg_s.at[pl.ds(start + i * CHUNK, CHUNK)])
        pltpu.sync_copy(rows.at[0, pl.ds(0, ROWS_MAIN - 4 * CHUNK)],
                        agg_s.at[pl.ds(start + 4 * CHUNK,
                                       ROWS_MAIN - 4 * CHUNK)])

    @pl.when(s == NS - 1)
    def _():
        for i in range(5):
            pltpu.sync_copy(rows.at[0],
                            agg_s.at[pl.ds(LAST_START + i * CHUNK, CHUNK)])
        pltpu.sync_copy(rows.at[0, pl.ds(0, PAD_ROWS)],
                        agg_s.at[pl.ds(LAST_START + 5 * CHUNK, PAD_ROWS)])

    plsc.subcore_barrier()

    def _make_round(src_idx_hbm):
        def _round(g, carry):
            base = g * NBUF
            di, dg, ds = [], [], []
            for b in range(NBUF):
                j = base + b
                di.append(
                    pltpu.async_copy(src_idx_hbm.at[s, j], idx_v[b],
                                     sem_i[b])
                )
            for b in range(NBUF):
                di[b].wait()
                dg.append(
                    pltpu.async_copy(x_hbm.at[idx_v[b].at[0]], rows.at[b],
                                     sem_g[b])
                )
            for b in range(NBUF):
                dg[b].wait()
                pltpu.sync_copy(rows.at[b], agg_s.at[idx_v[b].at[1]],
                                add=True)
            return carry
        return _round

    @pl.when(c == 0)
    def _():
        lax.fori_loop(0, K0 // NBUF, _make_round(idx0_hbm), 0)

    @pl.when(c == 1)
    def _():
        lax.fori_loop(0, K1 // NBUF, _make_round(idx1_hbm), 0)

    plsc.subcore_barrier()

    @pl.when(s < NS - 1)
    def _():
        pltpu.sync_copy(agg_s.at[pl.ds(start, ROWS_MAIN)],
                        parts_hbm.at[c, pl.ds(start, ROWS_MAIN)])

    @pl.when(s == NS - 1)
    def _():
        pltpu.sync_copy(agg_s.at[pl.ds(LAST_START, LAST_ROWS)],
                        parts_hbm.at[c, pl.ds(LAST_START, LAST_ROWS)])


def _tc_body(x_ref, p0_ref, p1_ref, wr_ref, wn_ref, b_ref, o_ref):
    agg = p0_ref[...] + p1_ref[...]
    acc = jnp.dot(x_ref[...], wr_ref[...], preferred_element_type=jnp.float32)
    acc = acc + jnp.dot(agg, wn_ref[...], preferred_element_type=jnp.float32)
    o_ref[...] = jnp.maximum(acc + b_ref[...], 0.0)


_ROW_BLK = 1000

_tc_finish = pl.pallas_call(
    _tc_body,
    grid=(N_NODES // _ROW_BLK,),
    in_specs=[
        pl.BlockSpec((_ROW_BLK, F), lambda i: (i, 0)),
        pl.BlockSpec((_ROW_BLK, F), lambda i: (i, 0)),
        pl.BlockSpec((_ROW_BLK, F), lambda i: (i, 0)),
        pl.BlockSpec((F, F), lambda i: (0, 0)),
        pl.BlockSpec((F, F), lambda i: (0, 0)),
        pl.BlockSpec((1, F), lambda i: (0, 0)),
    ],
    out_specs=pl.BlockSpec((_ROW_BLK, F), lambda i: (i, 0)),
    out_shape=jax.ShapeDtypeStruct((N_NODES, F), jnp.float32),
)


@jax.jit
def kernel(x, edge_index, W_root, W_nbr, b):
    ei = edge_index.astype(jnp.int32)
    pad = E_PAD - N_EDGES
    src = jnp.concatenate([ei[0], jnp.zeros((pad,), jnp.int32)])
    dst = jnp.concatenate([ei[1], jnp.full((pad,), N_NODES, jnp.int32)])
    idx0 = jnp.stack([src[:E_SPLIT].reshape(NS, K0, CHUNK),
                      dst[:E_SPLIT].reshape(NS, K0, CHUNK)], axis=2)
    idx1 = jnp.stack([src[E_SPLIT:].reshape(NS, K1, CHUNK),
                      dst[E_SPLIT:].reshape(NS, K1, CHUNK)], axis=2)
    parts = _sc_aggregate(x, idx0, idx1)
    return _tc_finish(x, parts[0], parts[1], W_root, W_nbr,
                      b.reshape(1, F))


# speedup vs baseline: 1.0239x; 1.0239x over previous
import functools

import jax
import jax.numpy as jnp
from jax import lax
from jax.experimental import pallas as pl
from jax.experimental.pallas import tpu as pltpu
from jax.experimental.pallas import tpu_sc as plsc

N_NODES = 10000
N_EDGES = 320000
F = 128

NC = 2
NS = 16
NW = NC * NS

CHUNK = 128
NBUF = 2
K0 = 116
K1 = 42
E_PAD = NS * (K0 + K1) * CHUNK
E_SPLIT = NS * K0 * CHUNK
PAD_ROWS = 8

ROWS_MAIN = 624
LAST_START = (NS - 1) * ROWS_MAIN
LAST_ROWS = N_NODES - LAST_START

_sc_mesh = plsc.VectorSubcoreMesh(core_axis_name="c", subcore_axis_name="s")


@functools.partial(
    pl.kernel,
    out_type=jax.ShapeDtypeStruct((NC, N_NODES, F), jnp.float32),
    mesh=_sc_mesh,
    scratch_types=[
        pltpu.VMEM_SHARED((N_NODES + PAD_ROWS, F), jnp.float32),
        [pltpu.VMEM((2, CHUNK), jnp.int32) for _ in range(NBUF)],
        pltpu.VMEM((NBUF, CHUNK, F), jnp.float32),
        [pltpu.SemaphoreType.DMA for _ in range(NBUF)],
        [pltpu.SemaphoreType.DMA for _ in range(NBUF)],
        [pltpu.SemaphoreType.DMA for _ in range(NBUF)],
    ],
)
def _sc_aggregate(x_hbm, idx0_hbm, idx1_hbm, parts_hbm,
                  agg_s, idx_v, rows, sem_i, sem_g, sem_s):
    c = lax.axis_index("c")
    s = lax.axis_index("s")

    start = pl.multiple_of(s * ROWS_MAIN, 8)

    zv = jnp.zeros((16,), jnp.float32)

    def _zrow(r, carry):
        for kk in range(F // 16):
            rows[0, r, pl.ds(kk * 16, 16)] = zv
        return carry

    lax.fori_loop(0, CHUNK, _zrow, 0)

    @pl.when(s < NS - 1)
    def _():
        for i in range(4):
            pltpu.sync_copy(rows.at[0],
                            agg_s.at[pl.ds(start + i * CHUNK, CHUNK)])
        pltpu.sync_copy(rows.at[0, pl.ds(0, ROWS_MAIN - 4 * CHUNK)],
                        agg_s.at[pl.ds(start + 4 * CHUNK,
                                       ROWS_MAIN - 4 * CHUNK)])

    @pl.when(s == NS - 1)
    def _():
        for i in range(5):
            pltpu.sync_copy(rows.at[0],
                            agg_s.at[pl.ds(LAST_START + i * CHUNK, CHUNK)])
        pltpu.sync_copy(rows.at[0, pl.ds(0, PAD_ROWS)],
                        agg_s.at[pl.ds(LAST_START + 5 * CHUNK, PAD_ROWS)])

    plsc.subcore_barrier()

    def _make_round(src_idx_hbm):
        def _round(g, carry):
            base = g * NBUF
            dg = []
            for b in range(NBUF):
                j = base + b
                pltpu.sync_copy(src_idx_hbm.at[s, j], idx_v[b])
                dg.append(
                    pltpu.async_copy(x_hbm.at[idx_v[b].at[0]], rows.at[b],
                                     sem_g[b])
                )
            for b in range(NBUF):
                dg[b].wait()
                pltpu.sync_copy(rows.at[b], agg_s.at[idx_v[b].at[1]],
                                add=True)
            return carry
        return _round

    @pl.when(c == 0)
    def _():
        lax.fori_loop(0, K0 // NBUF, _make_round(idx0_hbm), 0)

    @pl.when(c == 1)
    def _():
        lax.fori_loop(0, K1 // NBUF, _make_round(idx1_hbm), 0)

    plsc.subcore_barrier()

    @pl.when(s < NS - 1)
    def _():
        pltpu.sync_copy(agg_s.at[pl.ds(start, ROWS_MAIN)],
                        parts_hbm.at[c, pl.ds(start, ROWS_MAIN)])

    @pl.when(s == NS - 1)
    def _():
        pltpu.sync_copy(agg_s.at[pl.ds(LAST_START, LAST_ROWS)],
                        parts_hbm.at[c, pl.ds(LAST_START, LAST_ROWS)])


def _tc_body(x_ref, p0_ref, p1_ref, wr_ref, wn_ref, b_ref, o_ref):
    agg = p0_ref[...] + p1_ref[...]
    acc = jnp.dot(x_ref[...], wr_ref[...], preferred_element_type=jnp.float32)
    acc = acc + jnp.dot(agg, wn_ref[...], preferred_element_type=jnp.float32)
    o_ref[...] = jnp.maximum(acc + b_ref[...], 0.0)


_ROW_BLK = 1000

_tc_finish = pl.pallas_call(
    _tc_body,
    grid=(N_NODES // _ROW_BLK,),
    in_specs=[
        pl.BlockSpec((_ROW_BLK, F), lambda i: (i, 0)),
        pl.BlockSpec((_ROW_BLK, F), lambda i: (i, 0)),
        pl.BlockSpec((_ROW_BLK, F), lambda i: (i, 0)),
        pl.BlockSpec((F, F), lambda i: (0, 0)),
        pl.BlockSpec((F, F), lambda i: (0, 0)),
        pl.BlockSpec((1, F), lambda i: (0, 0)),
    ],
    out_specs=pl.BlockSpec((_ROW_BLK, F), lambda i: (i, 0)),
    out_shape=jax.ShapeDtypeStruct((N_NODES, F), jnp.float32),
)


@jax.jit
def kernel(x, edge_index, W_root, W_nbr, b):
    ei = edge_index.astype(jnp.int32)
    pad = E_PAD - N_EDGES
    src = jnp.concatenate([ei[0], jnp.zeros((pad,), jnp.int32)])
    dst = jnp.concatenate([ei[1], jnp.full((pad,), N_NODES, jnp.int32)])
    idx0 = jnp.stack([src[:E_SPLIT].reshape(NS, K0, CHUNK),
                      dst[:E_SPLIT].reshape(NS, K0, CHUNK)], axis=2)
    idx1 = jnp.stack([src[E_SPLIT:].reshape(NS, K1, CHUNK),
                      dst[E_SPLIT:].reshape(NS, K1, CHUNK)], axis=2)
    parts = _sc_aggregate(x, idx0, idx1)
    return _tc_finish(x, parts[0], parts[1], W_root, W_nbr,
                      b.reshape(1, F))
